# Initial kernel scaffold; baseline (speedup 1.0000x reference)
import jax
import jax.numpy as jnp
from jax.experimental import pallas as pl


def kernel(data, x, RWPE, adj_t, batch, W_rw, b_rw, W1s, b1s, gs, bes, W2s, b2s):
    raise NotImplementedError("write your pallas kernel here")


# speedup vs baseline: 1.9253x; 1.9253x over previous
import functools

import jax
import jax.numpy as jnp
from jax import lax
from jax.experimental import pallas as pl
from jax.experimental.pallas import tpu as pltpu
from jax.experimental.pallas import tpu_sc as plsc

N = 10000
E = 320000
D = 128
P = 20
G = 64

EPAD = 327680
NPAD = N + 16
CH = 128
RB = 1000

_SC_MESH = plsc.VectorSubcoreMesh(core_axis_name="c", subcore_axis_name="s")


def _fill_zeros(zbuf):
    z = jnp.zeros((16,), jnp.float32)

    def body(i, carry):
        r = i // (D // 16)
        c = (i % (D // 16)) * 16
        zbuf[r, pl.ds(c, 16)] = z
        return carry

    lax.fori_loop(0, 16 * (D // 16), body, 0)


def _zero_acc(acc, zbuf, sid):
    nrows = jnp.where(sid < 15, 640, 400)
    base = sid * 640

    def body(i, carry):
        @pl.when(i * 16 < nrows)
        def _():
            pltpu.sync_copy(zbuf, acc.at[pl.ds(base + i * 16, 16)])
        return carry

    lax.fori_loop(0, 40, body, 0)


def _edge_loop(tbl, srcb, dstb, acc, idx_s, idx_d, rows, sem, ebase, nch,
               src_off):
    def body(i, carry):
        b = ebase + i * CH
        pltpu.sync_copy(srcb.at[pl.ds(src_off + b, CH)], idx_s)
        pltpu.sync_copy(dstb.at[pl.ds(b, CH)], idx_d)
        pltpu.async_copy(tbl.at[idx_s], rows, sem).wait()
        pltpu.sync_copy(rows, acc.at[idx_d], add=True)
        return carry

    lax.fori_loop(0, nch, body, 0)


def _writeout(acc, out, sid, row_off):
    @pl.when(sid < 15)
    def _():
        rb = sid * 640
        pltpu.sync_copy(acc.at[pl.ds(rb, 640)],
                        out.at[pl.ds(row_off + rb, 640)])

    @pl.when(sid == 15)
    def _():
        pltpu.sync_copy(acc.at[pl.ds(9600, 400)],
                        out.at[pl.ds(row_off + 9600, 400)])


_SC_SCRATCH = [
    pltpu.VMEM((CH,), jnp.int32),
    pltpu.VMEM((CH,), jnp.int32),
    pltpu.VMEM((CH, D), jnp.float32),
    pltpu.VMEM((16, D), jnp.float32),
    pltpu.VMEM_SHARED((NPAD, D), jnp.float32),
    pltpu.SemaphoreType.DMA,
]


@functools.partial(
    pl.kernel,
    out_type=jax.ShapeDtypeStruct((2 * N, D), jnp.float32),
    mesh=_SC_MESH,
    scratch_types=_SC_SCRATCH,
)
def _agg_pair(tbl, srcb, dstb, out, idx_s, idx_d, rows, zbuf, acc, sem):
    cid = lax.axis_index("c")
    sid = lax.axis_index("s")
    _fill_zeros(zbuf)
    _zero_acc(acc, zbuf, sid)
    plsc.subcore_barrier()
    ep = EPAD // 16
    _edge_loop(tbl, srcb, dstb, acc, idx_s, idx_d, rows, sem,
               sid * ep, ep // CH, cid * EPAD)
    plsc.subcore_barrier()
    _writeout(acc, out, sid, cid * N)


@functools.partial(
    pl.kernel,
    out_type=jax.ShapeDtypeStruct((2 * N, D), jnp.float32),
    mesh=_SC_MESH,
    scratch_types=_SC_SCRATCH,
)
def _agg_single(tbl, srcb, dstb, out, idx_s, idx_d, rows, zbuf, acc, sem):
    cid = lax.axis_index("c")
    sid = lax.axis_index("s")
    _fill_zeros(zbuf)
    _zero_acc(acc, zbuf, sid)
    plsc.subcore_barrier()
    wid = sid * 2 + cid
    ep = EPAD // 32
    _edge_loop(tbl, srcb, dstb, acc, idx_s, idx_d, rows, sem,
               wid * ep, ep // CH, 0)
    plsc.subcore_barrier()
    _writeout(acc, out, sid, cid * N)


def _t1_body(x_ref, rwpe_ref, w_ref, b_ref, out_ref):
    rw = jnp.dot(rwpe_ref[...], w_ref[...],
                 preferred_element_type=jnp.float32) + b_ref[...]
    out_ref[0] = x_ref[...] + rw
    out_ref[1] = rw


_t1 = pl.pallas_call(
    _t1_body,
    grid=(N // RB,),
    in_specs=[
        pl.BlockSpec((RB, D), lambda i: (i, 0)),
        pl.BlockSpec((RB, D), lambda i: (i, 0)),
        pl.BlockSpec((D, D), lambda i: (0, 0)),
        pl.BlockSpec((1, D), lambda i: (0, 0)),
    ],
    out_specs=pl.BlockSpec((2, RB, D), lambda i: (0, i, 0)),
    out_shape=jax.ShapeDtypeStruct((2, N, D), jnp.float32),
)


def _mlp(z, w1, b1, w2, b2):
    z = jnp.maximum(jnp.dot(z, w1, preferred_element_type=jnp.float32) + b1,
                    0.0)
    return jnp.maximum(jnp.dot(z, w2, preferred_element_type=jnp.float32) + b2,
                       0.0)


def _t2_body(hrw_ref, agg_ref, w1h, b1h, w2h, b2h, w1r, b1r, w2r, b2r,
             out_ref):
    gh = _mlp(hrw_ref[0] + agg_ref[0], w1h[...], b1h[...], w2h[...], b2h[...])
    gr = _mlp(hrw_ref[1] + agg_ref[1], w1r[...], b1r[...], w2r[...], b2r[...])
    out_ref[0] = gh + gr
    out_ref[1] = gr


_t2 = pl.pallas_call(
    _t2_body,
    grid=(N // RB,),
    in_specs=[
        pl.BlockSpec((2, RB, D), lambda i: (0, i, 0)),
        pl.BlockSpec((2, RB, D), lambda i: (0, i, 0)),
    ] + [
        pl.BlockSpec((D, D), lambda i: (0, 0)),
        pl.BlockSpec((1, D), lambda i: (0, 0)),
        pl.BlockSpec((D, D), lambda i: (0, 0)),
        pl.BlockSpec((1, D), lambda i: (0, 0)),
    ] * 2,
    out_specs=pl.BlockSpec((2, RB, D), lambda i: (0, i, 0)),
    out_shape=jax.ShapeDtypeStruct((2, N, D), jnp.float32),
)


def _t3_body(h_ref, aggp_ref, w1, b1, w2, b2, s_ref, e_ref, out_ref):
    z = _mlp(h_ref[...] + aggp_ref[0] + aggp_ref[1],
             w1[...], b1[...], w2[...], b2[...])
    i = pl.program_id(0)
    col = lax.broadcasted_iota(jnp.int32, (G, RB), 1) + i * RB
    m = ((col >= s_ref[...]) & (col < e_ref[...])).astype(jnp.float32)
    seg = jnp.dot(m, z, preferred_element_type=jnp.float32)

    @pl.when(i == 0)
    def _():
        out_ref[...] = jnp.zeros_like(out_ref)

    out_ref[...] += seg


_t3 = pl.pallas_call(
    _t3_body,
    grid=(N // RB,),
    in_specs=[
        pl.BlockSpec((RB, D), lambda i: (i, 0)),
        pl.BlockSpec((2, RB, D), lambda i: (0, i, 0)),
        pl.BlockSpec((D, D), lambda i: (0, 0)),
        pl.BlockSpec((1, D), lambda i: (0, 0)),
        pl.BlockSpec((D, D), lambda i: (0, 0)),
        pl.BlockSpec((1, D), lambda i: (0, 0)),
        pl.BlockSpec((G, 1), lambda i: (0, 0)),
        pl.BlockSpec((G, 1), lambda i: (0, 0)),
    ],
    out_specs=pl.BlockSpec((G, D), lambda i: (0, 0)),
    out_shape=jax.ShapeDtypeStruct((G, D), jnp.float32),
)


def kernel(data, x, RWPE, adj_t, batch, W_rw, b_rw, W1s, b1s, gs, bes, W2s,
           b2s):
    src = adj_t[0]
    dst = adj_t[1]
    pad = EPAD - E
    src_p = jnp.concatenate([src, jnp.zeros((pad,), jnp.int32)])
    dst_p = jnp.concatenate([dst, jnp.full((pad,), N, jnp.int32)])
    src2 = jnp.concatenate([src_p, src_p + N])

    rwpe_p = jnp.pad(RWPE, ((0, 0), (0, D - P)))
    wrw_p = jnp.pad(W_rw.T, ((0, D - P), (0, 0)))

    scale = gs / jnp.sqrt(1.0 + 1e-5)
    W1e = W1s.transpose(0, 2, 1) * scale[:, None, :]
    b1e = (b1s * scale + bes).reshape(9, 1, D)
    W2T = W2s.transpose(0, 2, 1)
    b2r = b2s.reshape(9, 1, D)
    starts = batch[:G].reshape(G, 1).astype(jnp.int32)
    ends = batch[1:].reshape(G, 1).astype(jnp.int32)

    hrw = _t1(x, rwpe_p, wrw_p, b_rw.reshape(1, D))
    for k in range(4):
        agg = _agg_pair(hrw.reshape(2 * N, D), src2, dst_p).reshape(2, N, D)
        hrw = _t2(hrw, agg,
                  W1e[k], b1e[k], W2T[k], b2r[k],
                  W1e[5 + k], b1e[5 + k], W2T[5 + k], b2r[5 + k])
    h = hrw[0]
    aggp = _agg_single(h, src_p, dst_p).reshape(2, N, D)
    return _t3(h, aggp, W1e[4], b1e[4], W2T[4], b2r[4], starts, ends)
